# Initial kernel scaffold; baseline (speedup 1.0000x reference)
import jax
import jax.numpy as jnp
from jax.experimental import pallas as pl


def kernel(x, table):
    raise NotImplementedError("write your pallas kernel here")


# speedup vs baseline: 3.3042x; 3.3042x over previous
import functools

import jax
import jax.numpy as jnp
from jax import lax
from jax.experimental import pallas as pl
from jax.experimental.pallas import tpu as pltpu
from jax.experimental.pallas import tpu_sc as plsc

D_MODEL = 128
NUM_CORES = 2
NUM_SUBCORES = 16
NUM_WORKERS = NUM_CORES * NUM_SUBCORES

CHUNK = 128


@functools.lru_cache(maxsize=None)
def _make_gather(B: int):
    assert B % NUM_WORKERS == 0
    b_per_w = B // NUM_WORKERS
    assert b_per_w % CHUNK == 0
    n_chunks = b_per_w // CHUNK
    mesh = plsc.VectorSubcoreMesh(core_axis_name="c", subcore_axis_name="s")

    @functools.partial(
        pl.kernel,
        mesh=mesh,
        out_type=jax.ShapeDtypeStruct((B, D_MODEL), jnp.float32),
        scratch_types=[
            pltpu.VMEM((b_per_w,), jnp.int32),
            pltpu.VMEM((CHUNK, D_MODEL), jnp.float32),
            pltpu.SemaphoreType.DMA,
        ],
    )
    def grab(idx_hbm, table_hbm, out_hbm, idx_v, rows_v, sem):
        wid = lax.axis_index("s") * NUM_CORES + lax.axis_index("c")
        base = wid * b_per_w
        pltpu.sync_copy(idx_hbm.at[pl.ds(base, b_per_w)], idx_v)
        for c in range(n_chunks):
            pltpu.async_copy(
                table_hbm.at[idx_v.at[pl.ds(c * CHUNK, CHUNK)]], rows_v, sem
            ).wait()
            pltpu.sync_copy(rows_v, out_hbm.at[pl.ds(base + c * CHUNK, CHUNK)])

    return grab


def kernel(x, table):
    batch, seq = x.shape
    flat_idx = x.reshape(batch * seq).astype(jnp.int32)
    out = _make_gather(batch * seq)(flat_idx, table)
    return out.reshape(batch, seq, D_MODEL)


# speedup vs baseline: 3.7782x; 1.1435x over previous
import functools

import jax
import jax.numpy as jnp
from jax import lax
from jax.experimental import pallas as pl
from jax.experimental.pallas import tpu as pltpu
from jax.experimental.pallas import tpu_sc as plsc

D_MODEL = 128
NUM_CORES = 2
NUM_SUBCORES = 16
NUM_WORKERS = NUM_CORES * NUM_SUBCORES

CHUNK = 128
NBUF = 4
DEPTH = 2


@functools.lru_cache(maxsize=None)
def _make_gather(B: int):
    assert B % NUM_WORKERS == 0
    b_per_w = B // NUM_WORKERS
    assert b_per_w % CHUNK == 0
    n_chunks = b_per_w // CHUNK
    mesh = plsc.VectorSubcoreMesh(core_axis_name="c", subcore_axis_name="s")

    @functools.partial(
        pl.kernel,
        mesh=mesh,
        out_type=jax.ShapeDtypeStruct((B, D_MODEL), jnp.float32),
        scratch_types=[
            pltpu.VMEM((b_per_w,), jnp.int32),
            pltpu.VMEM((NBUF, CHUNK, D_MODEL), jnp.float32),
        ]
        + [pltpu.SemaphoreType.DMA] * (2 * NBUF),
    )
    def grab(idx_hbm, table_hbm, out_hbm, idx_v, rows_v, *sems):
        gsem, wsem = sems[:NBUF], sems[NBUF:]
        wid = lax.axis_index("s") * NUM_CORES + lax.axis_index("c")
        base = wid * b_per_w
        pltpu.sync_copy(idx_hbm.at[pl.ds(base, b_per_w)], idx_v)

        gathers = [None] * n_chunks
        writes = [None] * n_chunks

        def start_write(d):
            gathers[d].wait()
            writes[d] = pltpu.async_copy(
                rows_v.at[d % NBUF],
                out_hbm.at[pl.ds(base + d * CHUNK, CHUNK)],
                wsem[d % NBUF],
            )

        for c in range(n_chunks):
            b = c % NBUF
            if c >= NBUF:
                writes[c - NBUF].wait()
            gathers[c] = pltpu.async_copy(
                table_hbm.at[idx_v.at[pl.ds(c * CHUNK, CHUNK)]],
                rows_v.at[b],
                gsem[b],
            )
            if c - (DEPTH - 1) >= 0:
                start_write(c - (DEPTH - 1))
        for d in range(n_chunks - (DEPTH - 1), n_chunks):
            start_write(d)
        for d in range(max(0, n_chunks - NBUF), n_chunks):
            writes[d].wait()

    return grab


def kernel(x, table):
    batch, seq = x.shape
    flat_idx = x.reshape(batch * seq).astype(jnp.int32)
    out = _make_gather(batch * seq)(flat_idx, table)
    return out.reshape(batch, seq, D_MODEL)


# speedup vs baseline: 3.7948x; 1.0044x over previous
import functools

import jax
import jax.numpy as jnp
from jax import lax
from jax.experimental import pallas as pl
from jax.experimental.pallas import tpu as pltpu
from jax.experimental.pallas import tpu_sc as plsc

D_MODEL = 128
NUM_CORES = 2
NUM_SUBCORES = 16
NUM_WORKERS = NUM_CORES * NUM_SUBCORES

CHUNK = 128
NBUF = 6
DEPTH = 4


@functools.lru_cache(maxsize=None)
def _make_gather(B: int):
    assert B % NUM_WORKERS == 0
    b_per_w = B // NUM_WORKERS
    assert b_per_w % CHUNK == 0
    n_chunks = b_per_w // CHUNK
    mesh = plsc.VectorSubcoreMesh(core_axis_name="c", subcore_axis_name="s")

    @functools.partial(
        pl.kernel,
        mesh=mesh,
        out_type=jax.ShapeDtypeStruct((B, D_MODEL), jnp.float32),
        scratch_types=[
            pltpu.VMEM((b_per_w,), jnp.int32),
            pltpu.VMEM((NBUF, CHUNK, D_MODEL), jnp.float32),
        ]
        + [pltpu.SemaphoreType.DMA] * (2 * NBUF),
    )
    def grab(idx_hbm, table_hbm, out_hbm, idx_v, rows_v, *sems):
        gsem, wsem = sems[:NBUF], sems[NBUF:]
        wid = lax.axis_index("s") * NUM_CORES + lax.axis_index("c")
        base = wid * b_per_w
        pltpu.sync_copy(idx_hbm.at[pl.ds(base, b_per_w)], idx_v)

        gathers = [None] * n_chunks
        writes = [None] * n_chunks

        def start_write(d):
            gathers[d].wait()
            writes[d] = pltpu.async_copy(
                rows_v.at[d % NBUF],
                out_hbm.at[pl.ds(base + d * CHUNK, CHUNK)],
                wsem[d % NBUF],
            )

        for c in range(n_chunks):
            b = c % NBUF
            if c >= NBUF:
                writes[c - NBUF].wait()
            gathers[c] = pltpu.async_copy(
                table_hbm.at[idx_v.at[pl.ds(c * CHUNK, CHUNK)]],
                rows_v.at[b],
                gsem[b],
            )
            if c - (DEPTH - 1) >= 0:
                start_write(c - (DEPTH - 1))
        for d in range(n_chunks - (DEPTH - 1), n_chunks):
            start_write(d)
        for d in range(max(0, n_chunks - NBUF), n_chunks):
            writes[d].wait()

    return grab


def kernel(x, table):
    batch, seq = x.shape
    flat_idx = x.reshape(batch * seq).astype(jnp.int32)
    out = _make_gather(batch * seq)(flat_idx, table)
    return out.reshape(batch, seq, D_MODEL)
